# Initial kernel scaffold; baseline (speedup 1.0000x reference)
import jax
import jax.numpy as jnp
from jax.experimental import pallas as pl


def kernel(x, edge_index, batch, target, W1, b1, W2, b2, W3, b3, Wf1, bf1, Wf2, bf2):
    raise NotImplementedError("write your pallas kernel here")


# speedup vs baseline: 16.1651x; 16.1651x over previous
import functools

import jax
import jax.numpy as jnp
from jax import lax
from jax.experimental import pallas as pl
from jax.experimental.pallas import tpu as pltpu
from jax.experimental.pallas import tpu_sc as plsc

N = 10000
E = 320000
NC, NS = 2, 16
NW = NC * NS
EPW = E // NW
K = 80
NCHUNK = EPW // K
NGRP, GC = 5, 25
RPT = 624
RPL = N - RPT * (NS - 1)
F = 128

_mesh = plsc.VectorSubcoreMesh(
    core_axis_name="c", subcore_axis_name="s", num_cores=NC, num_subcores=NS
)


@functools.partial(
    pl.kernel,
    out_type=jax.ShapeDtypeStruct((NC, N, 16), jnp.float32),
    mesh=_mesh,
    scratch_types=[
        pltpu.VMEM_SHARED((N, 16), jnp.float32),
        pltpu.VMEM((NGRP, GC, K), jnp.int32),
        pltpu.VMEM((K, 16), jnp.float32),
        pltpu.VMEM((RPL, 16), jnp.float32),
    ],
)
def _deg_call(dst_hbm, out_hbm, acc, dstv, onesv, zbuf):
    cid = lax.axis_index("c")
    sid = lax.axis_index("s")
    wid = cid * NS + sid

    def fill(i, _):
        zbuf[i, :] = jnp.zeros((16,), jnp.float32)
        return 0

    lax.fori_loop(0, RPL, fill, 0)

    def fill1(i, _):
        onesv[i, :] = jnp.ones((16,), jnp.float32)
        return 0

    lax.fori_loop(0, K, fill1, 0)

    base = sid * RPT
    pltpu.sync_copy(zbuf, acc.at[pl.ds(base, RPL)])
    pltpu.sync_copy(dst_hbm.at[wid], dstv)
    plsc.subcore_barrier()

    def body(g, _):
        def inner(j, _):
            pltpu.sync_copy(onesv, acc.at[dstv.at[g, j]], add=True)
            return 0

        lax.fori_loop(0, GC, inner, 0)
        return 0

    lax.fori_loop(0, NGRP, body, 0)
    plsc.subcore_barrier()

    @pl.when(sid == NS - 1)
    def _():
        pltpu.sync_copy(acc.at[pl.ds(base, RPL)], out_hbm.at[cid, pl.ds(base, RPL)])

    @pl.when(sid != NS - 1)
    def _():
        pltpu.sync_copy(acc.at[pl.ds(base, RPT)], out_hbm.at[cid, pl.ds(base, RPT)])


@functools.partial(
    pl.kernel,
    out_type=jax.ShapeDtypeStruct((NC, N, F), jnp.float32),
    mesh=_mesh,
    scratch_types=[
        pltpu.VMEM_SHARED((N, F), jnp.float32),
        pltpu.VMEM((GC, K), jnp.int32),
        pltpu.VMEM((GC, K), jnp.int32),
        pltpu.VMEM((K, F), jnp.float32),
        pltpu.SemaphoreType.DMA,
    ],
)
def _agg_call(table_hbm, src_hbm, dst_hbm, out_hbm, acc, srcv, dstv, rows, sem):
    cid = lax.axis_index("c")
    sid = lax.axis_index("s")
    wid = cid * NS + sid

    def fill(i, _):
        for f in range(F // 16):
            rows[i, pl.ds(f * 16, 16)] = jnp.zeros((16,), jnp.float32)
        return 0

    lax.fori_loop(0, K, fill, 0)

    base = sid * RPT
    for r in range(0, RPL, K):
        pltpu.sync_copy(rows, acc.at[pl.ds(base + r, K)])

    plsc.subcore_barrier()

    def body(g, _):
        pltpu.sync_copy(src_hbm.at[wid, g], srcv)
        pltpu.sync_copy(dst_hbm.at[wid, g], dstv)

        def inner(j, _):
            pltpu.async_copy(table_hbm.at[srcv.at[j]], rows, sem).wait()
            pltpu.sync_copy(rows, acc.at[dstv.at[j]], add=True)
            return 0

        lax.fori_loop(0, GC, inner, 0)
        return 0

    lax.fori_loop(0, NGRP, body, 0)
    plsc.subcore_barrier()

    @pl.when(sid == NS - 1)
    def _():
        pltpu.sync_copy(acc.at[pl.ds(base, RPL)], out_hbm.at[cid, pl.ds(base, RPL)])

    @pl.when(sid != NS - 1)
    def _():
        pltpu.sync_copy(acc.at[pl.ds(base, RPT)], out_hbm.at[cid, pl.ds(base, RPT)])


_R = 1000


def _prologue_body(d0, d1, x, dinv_out, hs_out):
    deg = d0[:, 0:1] + d1[:, 0:1] + 1.0
    dv = lax.rsqrt(deg)
    dinv_out[...] = dv
    hs_out[...] = dv * x[...]


def _prologue(dacc, x):
    grid = (N // _R,)
    return pl.pallas_call(
        _prologue_body,
        grid=grid,
        in_specs=[
            pl.BlockSpec((_R, 16), lambda i: (i, 0)),
            pl.BlockSpec((_R, 16), lambda i: (i, 0)),
            pl.BlockSpec((_R, 128), lambda i: (i, 0)),
        ],
        out_specs=[
            pl.BlockSpec((_R, 1), lambda i: (i, 0)),
            pl.BlockSpec((_R, 128), lambda i: (i, 0)),
        ],
        out_shape=[
            jax.ShapeDtypeStruct((N, 1), jnp.float32),
            jax.ShapeDtypeStruct((N, 128), jnp.float32),
        ],
    )(dacc[0], dacc[1], x)


def _layer_body(s0, s1, hs, dv, w, b, h_out, hs_out):
    d = dv[...]
    m = d * (s0[...] + s1[...] + hs[...])
    z = jnp.dot(m, w[...], preferred_element_type=jnp.float32) + b[...]
    z = jnp.maximum(z, 0.0)
    h_out[...] = z
    hs_out[...] = d * z


def _layer(s0, s1, hs, dinv, W, b):
    fi = hs.shape[1]
    fo = W.shape[1]
    grid = (N // _R,)
    return pl.pallas_call(
        _layer_body,
        grid=grid,
        in_specs=[
            pl.BlockSpec((_R, fi), lambda i: (i, 0)),
            pl.BlockSpec((_R, fi), lambda i: (i, 0)),
            pl.BlockSpec((_R, fi), lambda i: (i, 0)),
            pl.BlockSpec((_R, 1), lambda i: (i, 0)),
            pl.BlockSpec((fi, fo), lambda i: (0, 0)),
            pl.BlockSpec((1, fo), lambda i: (0, 0)),
        ],
        out_specs=[
            pl.BlockSpec((_R, fo), lambda i: (i, 0)),
            pl.BlockSpec((_R, fo), lambda i: (i, 0)),
        ],
        out_shape=[
            jax.ShapeDtypeStruct((N, fo), jnp.float32),
            jax.ShapeDtypeStruct((N, fo), jnp.float32),
        ],
    )(s0, s1, hs, dinv, W, b)


def _pool_body(h, ids, wf1, bf1, wf2, bf2, out, pooled):
    i = pl.program_id(0)

    @pl.when(i == 0)
    def _():
        pooled[...] = jnp.full((64, 512), -jnp.inf, jnp.float32)

    hb = h[...]
    idb = ids[...]
    gmin = jnp.min(idb)
    gmax = jnp.max(idb)

    def body(g, _):
        @pl.when((g >= gmin) & (g <= gmax))
        def _():
            m = idb == g
            v = jnp.max(jnp.where(m, hb, -jnp.inf), axis=0, keepdims=True)
            pooled[pl.ds(g, 1), :] = jnp.maximum(pooled[pl.ds(g, 1), :], v)

        return 0

    lax.fori_loop(0, 64, body, 0)

    @pl.when(i == pl.num_programs(0) - 1)
    def _():
        p = pooled[...]
        g1 = jnp.dot(p, wf1[...], preferred_element_type=jnp.float32) + bf1[...]
        g1 = jnp.maximum(g1, 0.0)
        out[...] = jnp.dot(g1, wf2[...], preferred_element_type=jnp.float32) + bf2[...]


def _pool_mlp(h3, ids, Wf1, bf1, Wf2, bf2):
    grid = (N // _R,)
    return pl.pallas_call(
        _pool_body,
        grid=grid,
        in_specs=[
            pl.BlockSpec((_R, 512), lambda i: (i, 0)),
            pl.BlockSpec((_R, 1), lambda i: (i, 0)),
            pl.BlockSpec((512, 1024), lambda i: (0, 0)),
            pl.BlockSpec((1, 1024), lambda i: (0, 0)),
            pl.BlockSpec((1024, 128), lambda i: (0, 0)),
            pl.BlockSpec((1, 128), lambda i: (0, 0)),
        ],
        out_specs=pl.BlockSpec((64, 128), lambda i: (0, 0)),
        out_shape=jax.ShapeDtypeStruct((64, 128), jnp.float32),
        scratch_shapes=[pltpu.VMEM((64, 512), jnp.float32)],
    )(h3, ids, Wf1, bf1, Wf2, bf2)


def kernel(x, edge_index, batch, target, W1, b1, W2, b2, W3, b3, Wf1, bf1, Wf2, bf2):
    src = edge_index[0].astype(jnp.int32).reshape(NW, NGRP, GC, K)
    dst = edge_index[1].astype(jnp.int32).reshape(NW, NGRP, GC, K)

    dacc = _deg_call(dst)
    dinv, hs0 = _prologue(dacc, x)

    s1 = _agg_call(hs0, src, dst)
    h1, hs1 = _layer(s1[0], s1[1], hs0, dinv, W1, b1.reshape(1, -1))

    s2 = _agg_call(hs1, src, dst)
    h2, hs2 = _layer(s2[0], s2[1], hs1, dinv, W2, b2.reshape(1, -1))

    s3a = _agg_call(hs2[:, :F], src, dst)
    s3b = _agg_call(hs2[:, F:], src, dst)
    s3_0 = jnp.concatenate([s3a[0], s3b[0]], axis=1)
    s3_1 = jnp.concatenate([s3a[1], s3b[1]], axis=1)
    h3, _ = _layer(s3_0, s3_1, hs2, dinv, W3, b3.reshape(1, -1))

    return _pool_mlp(
        h3,
        batch.astype(jnp.int32).reshape(N, 1),
        Wf1,
        bf1.reshape(1, -1),
        Wf2,
        bf2.reshape(1, -1),
    )


# speedup vs baseline: 19.3897x; 1.1995x over previous
import functools

import jax
import jax.numpy as jnp
from jax import lax
from jax.experimental import pallas as pl
from jax.experimental.pallas import tpu as pltpu
from jax.experimental.pallas import tpu_sc as plsc

N = 10000
E = 320000
NC, NS = 2, 16
NW = NC * NS
EPW = E // NW
K = 80
NCHUNK = EPW // K
NGRP, GC = 5, 25
RPT = 624
RPL = N - RPT * (NS - 1)
F = 128

_mesh = plsc.VectorSubcoreMesh(
    core_axis_name="c", subcore_axis_name="s", num_cores=NC, num_subcores=NS
)


@functools.partial(
    pl.kernel,
    out_type=jax.ShapeDtypeStruct((NC, N, 16), jnp.float32),
    mesh=_mesh,
    scratch_types=[
        pltpu.VMEM_SHARED((N, 16), jnp.float32),
        pltpu.VMEM((NGRP, GC, K), jnp.int32),
        pltpu.VMEM((K, 16), jnp.float32),
        pltpu.VMEM((RPL, 16), jnp.float32),
    ],
)
def _deg_call(dst_hbm, out_hbm, acc, dstv, onesv, zbuf):
    cid = lax.axis_index("c")
    sid = lax.axis_index("s")
    wid = cid * NS + sid

    def fill(i, _):
        zbuf[i, :] = jnp.zeros((16,), jnp.float32)
        return 0

    lax.fori_loop(0, RPL, fill, 0)

    def fill1(i, _):
        onesv[i, :] = jnp.ones((16,), jnp.float32)
        return 0

    lax.fori_loop(0, K, fill1, 0)

    base = sid * RPT
    pltpu.sync_copy(zbuf, acc.at[pl.ds(base, RPL)])
    pltpu.sync_copy(dst_hbm.at[wid], dstv)
    plsc.subcore_barrier()

    def body(g, _):
        def inner(j, _):
            pltpu.sync_copy(onesv, acc.at[dstv.at[g, j]], add=True)
            return 0

        lax.fori_loop(0, GC, inner, 0)
        return 0

    lax.fori_loop(0, NGRP, body, 0)
    plsc.subcore_barrier()

    @pl.when(sid == NS - 1)
    def _():
        pltpu.sync_copy(acc.at[pl.ds(base, RPL)], out_hbm.at[cid, pl.ds(base, RPL)])

    @pl.when(sid != NS - 1)
    def _():
        pltpu.sync_copy(acc.at[pl.ds(base, RPT)], out_hbm.at[cid, pl.ds(base, RPT)])


@functools.partial(
    pl.kernel,
    out_type=jax.ShapeDtypeStruct((NC, N, F), jnp.float32),
    mesh=_mesh,
    scratch_types=[
        pltpu.VMEM_SHARED((N, F), jnp.float32),
        pltpu.VMEM((GC, K), jnp.int32),
        pltpu.VMEM((GC, K), jnp.int32),
        pltpu.VMEM((K, F), jnp.float32),
        pltpu.VMEM((K, F), jnp.float32),
        pltpu.SemaphoreType.DMA,
        pltpu.SemaphoreType.DMA,
    ],
)
def _agg_call(table_hbm, src_hbm, dst_hbm, out_hbm, acc, srcv, dstv, rows, rows_b, sem, sem_b):
    cid = lax.axis_index("c")
    sid = lax.axis_index("s")
    wid = cid * NS + sid

    def fill(i, _):
        for f in range(F // 16):
            rows[i, pl.ds(f * 16, 16)] = jnp.zeros((16,), jnp.float32)
        return 0

    lax.fori_loop(0, K, fill, 0)

    base = sid * RPT
    for r in range(0, RPL, K):
        pltpu.sync_copy(rows, acc.at[pl.ds(base + r, K)])

    plsc.subcore_barrier()

    half = GC // 2

    def body(g, _):
        pltpu.sync_copy(src_hbm.at[wid, g], srcv)
        pltpu.sync_copy(dst_hbm.at[wid, g], dstv)

        def inner(j, _):
            a = pltpu.async_copy(table_hbm.at[srcv.at[j]], rows, sem)
            b = pltpu.async_copy(table_hbm.at[srcv.at[j + half]], rows_b, sem_b)
            a.wait()
            pltpu.sync_copy(rows, acc.at[dstv.at[j]], add=True)
            b.wait()
            pltpu.sync_copy(rows_b, acc.at[dstv.at[j + half]], add=True)
            return 0

        lax.fori_loop(0, half, inner, 0)
        pltpu.async_copy(table_hbm.at[srcv.at[GC - 1]], rows, sem).wait()
        pltpu.sync_copy(rows, acc.at[dstv.at[GC - 1]], add=True)
        return 0

    lax.fori_loop(0, NGRP, body, 0)
    plsc.subcore_barrier()

    @pl.when(sid == NS - 1)
    def _():
        pltpu.sync_copy(acc.at[pl.ds(base, RPL)], out_hbm.at[cid, pl.ds(base, RPL)])

    @pl.when(sid != NS - 1)
    def _():
        pltpu.sync_copy(acc.at[pl.ds(base, RPT)], out_hbm.at[cid, pl.ds(base, RPT)])


_R = 1000


def _prologue_body(d0, d1, x, dinv_out, hs_out):
    deg = d0[:, 0:1] + d1[:, 0:1] + 1.0
    dv = lax.rsqrt(deg)
    dinv_out[...] = dv
    hs_out[...] = dv * x[...]


def _prologue(dacc, x):
    grid = (N // _R,)
    return pl.pallas_call(
        _prologue_body,
        grid=grid,
        in_specs=[
            pl.BlockSpec((_R, 16), lambda i: (i, 0)),
            pl.BlockSpec((_R, 16), lambda i: (i, 0)),
            pl.BlockSpec((_R, 128), lambda i: (i, 0)),
        ],
        out_specs=[
            pl.BlockSpec((_R, 1), lambda i: (i, 0)),
            pl.BlockSpec((_R, 128), lambda i: (i, 0)),
        ],
        out_shape=[
            jax.ShapeDtypeStruct((N, 1), jnp.float32),
            jax.ShapeDtypeStruct((N, 128), jnp.float32),
        ],
    )(dacc[0], dacc[1], x)


def _layer_body(s0, s1, hs, dv, w, b, h_out, hs_out):
    d = dv[...]
    m = d * (s0[...] + s1[...] + hs[...])
    z = jnp.dot(m, w[...], preferred_element_type=jnp.float32) + b[...]
    z = jnp.maximum(z, 0.0)
    h_out[...] = z
    hs_out[...] = d * z


def _layer(s0, s1, hs, dinv, W, b):
    fi = hs.shape[1]
    fo = W.shape[1]
    grid = (N // _R,)
    return pl.pallas_call(
        _layer_body,
        grid=grid,
        in_specs=[
            pl.BlockSpec((_R, fi), lambda i: (i, 0)),
            pl.BlockSpec((_R, fi), lambda i: (i, 0)),
            pl.BlockSpec((_R, fi), lambda i: (i, 0)),
            pl.BlockSpec((_R, 1), lambda i: (i, 0)),
            pl.BlockSpec((fi, fo), lambda i: (0, 0)),
            pl.BlockSpec((1, fo), lambda i: (0, 0)),
        ],
        out_specs=[
            pl.BlockSpec((_R, fo), lambda i: (i, 0)),
            pl.BlockSpec((_R, fo), lambda i: (i, 0)),
        ],
        out_shape=[
            jax.ShapeDtypeStruct((N, fo), jnp.float32),
            jax.ShapeDtypeStruct((N, fo), jnp.float32),
        ],
    )(s0, s1, hs, dinv, W, b)


def _pool_body(h, ids, wf1, bf1, wf2, bf2, out, pooled):
    i = pl.program_id(0)

    @pl.when(i == 0)
    def _():
        pooled[...] = jnp.full((64, 512), -jnp.inf, jnp.float32)

    hb = h[...]
    idb = ids[...]
    gmin = jnp.min(idb)
    gmax = jnp.max(idb)

    def body(g, _):
        @pl.when((g >= gmin) & (g <= gmax))
        def _():
            m = idb == g
            v = jnp.max(jnp.where(m, hb, -jnp.inf), axis=0, keepdims=True)
            pooled[pl.ds(g, 1), :] = jnp.maximum(pooled[pl.ds(g, 1), :], v)

        return 0

    lax.fori_loop(0, 64, body, 0)

    @pl.when(i == pl.num_programs(0) - 1)
    def _():
        p = pooled[...]
        g1 = jnp.dot(p, wf1[...], preferred_element_type=jnp.float32) + bf1[...]
        g1 = jnp.maximum(g1, 0.0)
        out[...] = jnp.dot(g1, wf2[...], preferred_element_type=jnp.float32) + bf2[...]


def _pool_mlp(h3, ids, Wf1, bf1, Wf2, bf2):
    grid = (N // _R,)
    return pl.pallas_call(
        _pool_body,
        grid=grid,
        in_specs=[
            pl.BlockSpec((_R, 512), lambda i: (i, 0)),
            pl.BlockSpec((_R, 1), lambda i: (i, 0)),
            pl.BlockSpec((512, 1024), lambda i: (0, 0)),
            pl.BlockSpec((1, 1024), lambda i: (0, 0)),
            pl.BlockSpec((1024, 128), lambda i: (0, 0)),
            pl.BlockSpec((1, 128), lambda i: (0, 0)),
        ],
        out_specs=pl.BlockSpec((64, 128), lambda i: (0, 0)),
        out_shape=jax.ShapeDtypeStruct((64, 128), jnp.float32),
        scratch_shapes=[pltpu.VMEM((64, 512), jnp.float32)],
    )(h3, ids, Wf1, bf1, Wf2, bf2)


def kernel(x, edge_index, batch, target, W1, b1, W2, b2, W3, b3, Wf1, bf1, Wf2, bf2):
    src = edge_index[0].astype(jnp.int32).reshape(NW, NGRP, GC, K)
    dst = edge_index[1].astype(jnp.int32).reshape(NW, NGRP, GC, K)

    dacc = _deg_call(dst)
    dinv, hs0 = _prologue(dacc, x)

    s1 = _agg_call(hs0, src, dst)
    h1, hs1 = _layer(s1[0], s1[1], hs0, dinv, W1, b1.reshape(1, -1))

    s2 = _agg_call(hs1, src, dst)
    h2, hs2 = _layer(s2[0], s2[1], hs1, dinv, W2, b2.reshape(1, -1))

    s3a = _agg_call(hs2[:, :F], src, dst)
    s3b = _agg_call(hs2[:, F:], src, dst)
    s3_0 = jnp.concatenate([s3a[0], s3b[0]], axis=1)
    s3_1 = jnp.concatenate([s3a[1], s3b[1]], axis=1)
    h3, _ = _layer(s3_0, s3_1, hs2, dinv, W3, b3.reshape(1, -1))

    return _pool_mlp(
        h3,
        batch.astype(jnp.int32).reshape(N, 1),
        Wf1,
        bf1.reshape(1, -1),
        Wf2,
        bf2.reshape(1, -1),
    )


# speedup vs baseline: 20.4034x; 1.0523x over previous
import functools

import jax
import jax.numpy as jnp
from jax import lax
from jax.experimental import pallas as pl
from jax.experimental.pallas import tpu as pltpu
from jax.experimental.pallas import tpu_sc as plsc

N = 10000
E = 320000
NC, NS = 2, 16
NW = NC * NS
EPW = E // NW
K = 80
NCHUNK = EPW // K
NGRP, GC = 5, 25
RPT = 624
RPL = N - RPT * (NS - 1)
F = 128

_mesh = plsc.VectorSubcoreMesh(
    core_axis_name="c", subcore_axis_name="s", num_cores=NC, num_subcores=NS
)


@functools.partial(
    pl.kernel,
    out_type=jax.ShapeDtypeStruct((NC, N, 16), jnp.float32),
    mesh=_mesh,
    scratch_types=[
        pltpu.VMEM_SHARED((N, 16), jnp.float32),
        pltpu.VMEM((NGRP, GC, K), jnp.int32),
        pltpu.VMEM((K, 16), jnp.float32),
        pltpu.VMEM((RPL, 16), jnp.float32),
    ],
)
def _deg_call(dst_hbm, out_hbm, acc, dstv, onesv, zbuf):
    cid = lax.axis_index("c")
    sid = lax.axis_index("s")
    wid = cid * NS + sid

    def fill(i, _):
        zbuf[i, :] = jnp.zeros((16,), jnp.float32)
        return 0

    lax.fori_loop(0, RPL, fill, 0)

    def fill1(i, _):
        onesv[i, :] = jnp.ones((16,), jnp.float32)
        return 0

    lax.fori_loop(0, K, fill1, 0)

    base = sid * RPT
    pltpu.sync_copy(zbuf, acc.at[pl.ds(base, RPL)])
    pltpu.sync_copy(dst_hbm.at[wid], dstv)
    plsc.subcore_barrier()

    def body(g, _):
        def inner(j, _):
            pltpu.sync_copy(onesv, acc.at[dstv.at[g, j]], add=True)
            return 0

        lax.fori_loop(0, GC, inner, 0)
        return 0

    lax.fori_loop(0, NGRP, body, 0)
    plsc.subcore_barrier()

    @pl.when(sid == NS - 1)
    def _():
        pltpu.sync_copy(acc.at[pl.ds(base, RPL)], out_hbm.at[cid, pl.ds(base, RPL)])

    @pl.when(sid != NS - 1)
    def _():
        pltpu.sync_copy(acc.at[pl.ds(base, RPT)], out_hbm.at[cid, pl.ds(base, RPT)])


@functools.partial(
    pl.kernel,
    out_type=jax.ShapeDtypeStruct((NC, N, F), jnp.float32),
    mesh=_mesh,
    scratch_types=[
        pltpu.VMEM_SHARED((N, F), jnp.float32),
        pltpu.VMEM((GC, K), jnp.int32),
        pltpu.VMEM((GC, K), jnp.int32),
        pltpu.VMEM((K, F), jnp.float32),
        pltpu.VMEM((K, F), jnp.float32),
        pltpu.VMEM((K, F), jnp.float32),
        pltpu.VMEM((K, F), jnp.float32),
        pltpu.SemaphoreType.DMA,
        pltpu.SemaphoreType.DMA,
        pltpu.SemaphoreType.DMA,
        pltpu.SemaphoreType.DMA,
    ],
)
def _agg_call(
    table_hbm, src_hbm, dst_hbm, out_hbm, acc, srcv, dstv,
    rows, rows_b, rows_c, rows_d, sem, sem_b, sem_c, sem_d,
):
    cid = lax.axis_index("c")
    sid = lax.axis_index("s")
    wid = cid * NS + sid

    def fill(i, _):
        for f in range(F // 16):
            rows[i, pl.ds(f * 16, 16)] = jnp.zeros((16,), jnp.float32)
        return 0

    lax.fori_loop(0, K, fill, 0)

    base = sid * RPT
    for r in range(0, RPL, K):
        pltpu.sync_copy(rows, acc.at[pl.ds(base + r, K)])

    plsc.subcore_barrier()

    q = GC // 4

    def body(g, _):
        pltpu.sync_copy(src_hbm.at[wid, g], srcv)
        pltpu.sync_copy(dst_hbm.at[wid, g], dstv)

        def inner(j, _):
            a = pltpu.async_copy(table_hbm.at[srcv.at[j]], rows, sem)
            b = pltpu.async_copy(table_hbm.at[srcv.at[j + q]], rows_b, sem_b)
            c = pltpu.async_copy(table_hbm.at[srcv.at[j + 2 * q]], rows_c, sem_c)
            d = pltpu.async_copy(table_hbm.at[srcv.at[j + 3 * q]], rows_d, sem_d)
            a.wait()
            pltpu.sync_copy(rows, acc.at[dstv.at[j]], add=True)
            b.wait()
            pltpu.sync_copy(rows_b, acc.at[dstv.at[j + q]], add=True)
            c.wait()
            pltpu.sync_copy(rows_c, acc.at[dstv.at[j + 2 * q]], add=True)
            d.wait()
            pltpu.sync_copy(rows_d, acc.at[dstv.at[j + 3 * q]], add=True)
            return 0

        lax.fori_loop(0, q, inner, 0)
        pltpu.async_copy(table_hbm.at[srcv.at[GC - 1]], rows, sem).wait()
        pltpu.sync_copy(rows, acc.at[dstv.at[GC - 1]], add=True)
        return 0

    lax.fori_loop(0, NGRP, body, 0)
    plsc.subcore_barrier()

    @pl.when(sid == NS - 1)
    def _():
        pltpu.sync_copy(acc.at[pl.ds(base, RPL)], out_hbm.at[cid, pl.ds(base, RPL)])

    @pl.when(sid != NS - 1)
    def _():
        pltpu.sync_copy(acc.at[pl.ds(base, RPT)], out_hbm.at[cid, pl.ds(base, RPT)])


_R = 1000


def _prologue_body(d0, d1, x, dinv_out, hs_out):
    deg = d0[:, 0:1] + d1[:, 0:1] + 1.0
    dv = lax.rsqrt(deg)
    dinv_out[...] = dv
    hs_out[...] = dv * x[...]


def _prologue(dacc, x):
    grid = (N // _R,)
    return pl.pallas_call(
        _prologue_body,
        grid=grid,
        in_specs=[
            pl.BlockSpec((_R, 16), lambda i: (i, 0)),
            pl.BlockSpec((_R, 16), lambda i: (i, 0)),
            pl.BlockSpec((_R, 128), lambda i: (i, 0)),
        ],
        out_specs=[
            pl.BlockSpec((_R, 1), lambda i: (i, 0)),
            pl.BlockSpec((_R, 128), lambda i: (i, 0)),
        ],
        out_shape=[
            jax.ShapeDtypeStruct((N, 1), jnp.float32),
            jax.ShapeDtypeStruct((N, 128), jnp.float32),
        ],
    )(dacc[0], dacc[1], x)


def _layer_body(s0, s1, hs, dv, w, b, h_out, hs_out):
    d = dv[...]
    m = d * (s0[...] + s1[...] + hs[...])
    z = jnp.dot(m, w[...], preferred_element_type=jnp.float32) + b[...]
    z = jnp.maximum(z, 0.0)
    h_out[...] = z
    hs_out[...] = d * z


def _layer(s0, s1, hs, dinv, W, b):
    fi = hs.shape[1]
    fo = W.shape[1]
    grid = (N // _R,)
    return pl.pallas_call(
        _layer_body,
        grid=grid,
        in_specs=[
            pl.BlockSpec((_R, fi), lambda i: (i, 0)),
            pl.BlockSpec((_R, fi), lambda i: (i, 0)),
            pl.BlockSpec((_R, fi), lambda i: (i, 0)),
            pl.BlockSpec((_R, 1), lambda i: (i, 0)),
            pl.BlockSpec((fi, fo), lambda i: (0, 0)),
            pl.BlockSpec((1, fo), lambda i: (0, 0)),
        ],
        out_specs=[
            pl.BlockSpec((_R, fo), lambda i: (i, 0)),
            pl.BlockSpec((_R, fo), lambda i: (i, 0)),
        ],
        out_shape=[
            jax.ShapeDtypeStruct((N, fo), jnp.float32),
            jax.ShapeDtypeStruct((N, fo), jnp.float32),
        ],
    )(s0, s1, hs, dinv, W, b)


def _pool_body(h, ids, wf1, bf1, wf2, bf2, out, pooled):
    i = pl.program_id(0)

    @pl.when(i == 0)
    def _():
        pooled[...] = jnp.full((64, 512), -jnp.inf, jnp.float32)

    hb = h[...]
    idb = ids[...]
    gmin = jnp.min(idb)
    gmax = jnp.max(idb)

    def body(g, _):
        @pl.when((g >= gmin) & (g <= gmax))
        def _():
            m = idb == g
            v = jnp.max(jnp.where(m, hb, -jnp.inf), axis=0, keepdims=True)
            pooled[pl.ds(g, 1), :] = jnp.maximum(pooled[pl.ds(g, 1), :], v)

        return 0

    lax.fori_loop(0, 64, body, 0)

    @pl.when(i == pl.num_programs(0) - 1)
    def _():
        p = pooled[...]
        g1 = jnp.dot(p, wf1[...], preferred_element_type=jnp.float32) + bf1[...]
        g1 = jnp.maximum(g1, 0.0)
        out[...] = jnp.dot(g1, wf2[...], preferred_element_type=jnp.float32) + bf2[...]


def _pool_mlp(h3, ids, Wf1, bf1, Wf2, bf2):
    grid = (N // _R,)
    return pl.pallas_call(
        _pool_body,
        grid=grid,
        in_specs=[
            pl.BlockSpec((_R, 512), lambda i: (i, 0)),
            pl.BlockSpec((_R, 1), lambda i: (i, 0)),
            pl.BlockSpec((512, 1024), lambda i: (0, 0)),
            pl.BlockSpec((1, 1024), lambda i: (0, 0)),
            pl.BlockSpec((1024, 128), lambda i: (0, 0)),
            pl.BlockSpec((1, 128), lambda i: (0, 0)),
        ],
        out_specs=pl.BlockSpec((64, 128), lambda i: (0, 0)),
        out_shape=jax.ShapeDtypeStruct((64, 128), jnp.float32),
        scratch_shapes=[pltpu.VMEM((64, 512), jnp.float32)],
    )(h3, ids, Wf1, bf1, Wf2, bf2)


def kernel(x, edge_index, batch, target, W1, b1, W2, b2, W3, b3, Wf1, bf1, Wf2, bf2):
    src = edge_index[0].astype(jnp.int32).reshape(NW, NGRP, GC, K)
    dst = edge_index[1].astype(jnp.int32).reshape(NW, NGRP, GC, K)

    dacc = _deg_call(dst)
    dinv, hs0 = _prologue(dacc, x)

    s1 = _agg_call(hs0, src, dst)
    h1, hs1 = _layer(s1[0], s1[1], hs0, dinv, W1, b1.reshape(1, -1))

    s2 = _agg_call(hs1, src, dst)
    h2, hs2 = _layer(s2[0], s2[1], hs1, dinv, W2, b2.reshape(1, -1))

    s3a = _agg_call(hs2[:, :F], src, dst)
    s3b = _agg_call(hs2[:, F:], src, dst)
    s3_0 = jnp.concatenate([s3a[0], s3b[0]], axis=1)
    s3_1 = jnp.concatenate([s3a[1], s3b[1]], axis=1)
    h3, _ = _layer(s3_0, s3_1, hs2, dinv, W3, b3.reshape(1, -1))

    return _pool_mlp(
        h3,
        batch.astype(jnp.int32).reshape(N, 1),
        Wf1,
        bf1.reshape(1, -1),
        Wf2,
        bf2.reshape(1, -1),
    )


# speedup vs baseline: 20.7616x; 1.0176x over previous
import functools

import jax
import jax.numpy as jnp
from jax import lax
from jax.experimental import pallas as pl
from jax.experimental.pallas import tpu as pltpu
from jax.experimental.pallas import tpu_sc as plsc

N = 10000
E = 320000
NC, NS = 2, 16
NW = NC * NS
EPW = E // NW
K = 80
NCHUNK = EPW // K
NGRP, GC = 5, 25
RPT = 624
RPL = N - RPT * (NS - 1)
F = 128

_mesh = plsc.VectorSubcoreMesh(
    core_axis_name="c", subcore_axis_name="s", num_cores=NC, num_subcores=NS
)


@functools.partial(
    pl.kernel,
    out_type=jax.ShapeDtypeStruct((NC, N, 16), jnp.float32),
    mesh=_mesh,
    scratch_types=[
        pltpu.VMEM_SHARED((N, 16), jnp.float32),
        pltpu.VMEM((NGRP, GC, K), jnp.int32),
        pltpu.VMEM((K, 16), jnp.float32),
        pltpu.VMEM((RPL, 16), jnp.float32),
    ],
)
def _deg_call(dst_hbm, out_hbm, acc, dstv, onesv, zbuf):
    cid = lax.axis_index("c")
    sid = lax.axis_index("s")
    wid = cid * NS + sid

    def fill(i, _):
        zbuf[i, :] = jnp.zeros((16,), jnp.float32)
        return 0

    lax.fori_loop(0, RPL, fill, 0)

    def fill1(i, _):
        onesv[i, :] = jnp.ones((16,), jnp.float32)
        return 0

    lax.fori_loop(0, K, fill1, 0)

    base = sid * RPT
    pltpu.sync_copy(zbuf, acc.at[pl.ds(base, RPL)])
    pltpu.sync_copy(dst_hbm.at[wid], dstv)
    plsc.subcore_barrier()

    def body(g, _):
        def inner(j, _):
            pltpu.sync_copy(onesv, acc.at[dstv.at[g, j]], add=True)
            return 0

        lax.fori_loop(0, GC, inner, 0)
        return 0

    lax.fori_loop(0, NGRP, body, 0)
    plsc.subcore_barrier()

    @pl.when(sid == NS - 1)
    def _():
        pltpu.sync_copy(acc.at[pl.ds(base, RPL)], out_hbm.at[cid, pl.ds(base, RPL)])

    @pl.when(sid != NS - 1)
    def _():
        pltpu.sync_copy(acc.at[pl.ds(base, RPT)], out_hbm.at[cid, pl.ds(base, RPT)])


@functools.partial(
    pl.kernel,
    out_type=jax.ShapeDtypeStruct((NC, N, F), jnp.float32),
    mesh=_mesh,
    scratch_types=[
        pltpu.VMEM_SHARED((N, F), jnp.float32),
        pltpu.VMEM((GC, K), jnp.int32),
        pltpu.VMEM((GC, K), jnp.int32),
        pltpu.VMEM((K, F), jnp.float32),
        pltpu.VMEM((K, F), jnp.float32),
        pltpu.VMEM((K, F), jnp.float32),
        pltpu.VMEM((K, F), jnp.float32),
        pltpu.SemaphoreType.DMA,
        pltpu.SemaphoreType.DMA,
        pltpu.SemaphoreType.DMA,
        pltpu.SemaphoreType.DMA,
    ],
)
def _agg_call(
    table_hbm, src_hbm, dst_hbm, out_hbm, acc, srcv, dstv,
    rows, rows_b, rows_c, rows_d, sem, sem_b, sem_c, sem_d,
):
    cid = lax.axis_index("c")
    sid = lax.axis_index("s")
    wid = cid * NS + sid

    def fill(i, _):
        for f in range(F // 16):
            rows[i, pl.ds(f * 16, 16)] = jnp.zeros((16,), jnp.float32)
        return 0

    lax.fori_loop(0, K, fill, 0)

    base = sid * RPT
    for r in range(0, RPL, K):
        pltpu.sync_copy(rows, acc.at[pl.ds(base + r, K)])

    plsc.subcore_barrier()

    q = GC // 4

    def body(g, _):
        pltpu.sync_copy(src_hbm.at[wid, g], srcv)
        pltpu.sync_copy(dst_hbm.at[wid, g], dstv)

        def inner(j, _):
            a = pltpu.async_copy(table_hbm.at[srcv.at[j]], rows, sem)
            b = pltpu.async_copy(table_hbm.at[srcv.at[j + q]], rows_b, sem_b)
            c = pltpu.async_copy(table_hbm.at[srcv.at[j + 2 * q]], rows_c, sem_c)
            d = pltpu.async_copy(table_hbm.at[srcv.at[j + 3 * q]], rows_d, sem_d)
            a.wait()
            pltpu.sync_copy(rows, acc.at[dstv.at[j]], add=True)
            b.wait()
            pltpu.sync_copy(rows_b, acc.at[dstv.at[j + q]], add=True)
            c.wait()
            pltpu.sync_copy(rows_c, acc.at[dstv.at[j + 2 * q]], add=True)
            d.wait()
            pltpu.sync_copy(rows_d, acc.at[dstv.at[j + 3 * q]], add=True)
            return 0

        lax.fori_loop(0, q, inner, 0)
        pltpu.async_copy(table_hbm.at[srcv.at[GC - 1]], rows, sem).wait()
        pltpu.sync_copy(rows, acc.at[dstv.at[GC - 1]], add=True)
        return 0

    lax.fori_loop(0, NGRP, body, 0)
    plsc.subcore_barrier()

    @pl.when(sid == NS - 1)
    def _():
        pltpu.sync_copy(acc.at[pl.ds(base, RPL)], out_hbm.at[cid, pl.ds(base, RPL)])

    @pl.when(sid != NS - 1)
    def _():
        pltpu.sync_copy(acc.at[pl.ds(base, RPT)], out_hbm.at[cid, pl.ds(base, RPT)])


_R = 1000


def _prologue_body(d0, d1, x, dinv_out, hs_out):
    deg = d0[:, 0:1] + d1[:, 0:1] + 1.0
    dv = lax.rsqrt(deg)
    dinv_out[...] = dv
    hs_out[...] = dv * x[...]


def _prologue(dacc, x):
    grid = (N // _R,)
    return pl.pallas_call(
        _prologue_body,
        grid=grid,
        in_specs=[
            pl.BlockSpec((_R, 16), lambda i: (i, 0)),
            pl.BlockSpec((_R, 16), lambda i: (i, 0)),
            pl.BlockSpec((_R, 128), lambda i: (i, 0)),
        ],
        out_specs=[
            pl.BlockSpec((_R, 1), lambda i: (i, 0)),
            pl.BlockSpec((_R, 128), lambda i: (i, 0)),
        ],
        out_shape=[
            jax.ShapeDtypeStruct((N, 1), jnp.float32),
            jax.ShapeDtypeStruct((N, 128), jnp.float32),
        ],
    )(dacc[0], dacc[1], x)


def _layer_hs_body(s0, s1, hs, dv, w, b, hs_out):
    d = dv[...]
    m = d * (s0[...] + s1[...] + hs[...])
    z = jnp.dot(m, w[...], preferred_element_type=jnp.float32) + b[...]
    hs_out[...] = d * jnp.maximum(z, 0.0)


def _layer_hs(s0, s1, hs, dinv, W, b):
    fi = hs.shape[1]
    fo = W.shape[1]
    grid = (N // _R,)
    return pl.pallas_call(
        _layer_hs_body,
        grid=grid,
        in_specs=[
            pl.BlockSpec((_R, fi), lambda i: (i, 0)),
            pl.BlockSpec((_R, fi), lambda i: (i, 0)),
            pl.BlockSpec((_R, fi), lambda i: (i, 0)),
            pl.BlockSpec((_R, 1), lambda i: (i, 0)),
            pl.BlockSpec((fi, fo), lambda i: (0, 0)),
            pl.BlockSpec((1, fo), lambda i: (0, 0)),
        ],
        out_specs=pl.BlockSpec((_R, fo), lambda i: (i, 0)),
        out_shape=jax.ShapeDtypeStruct((N, fo), jnp.float32),
    )(s0, s1, hs, dinv, W, b)


def _layer_hs2_body(s0, s1, hs, dv, w, b, a_out, b_out):
    d = dv[...]
    m = d * (s0[...] + s1[...] + hs[...])
    z = jnp.dot(m, w[...], preferred_element_type=jnp.float32) + b[...]
    z = d * jnp.maximum(z, 0.0)
    a_out[...] = z[:, :F]
    b_out[...] = z[:, F:]


def _layer_hs2(s0, s1, hs, dinv, W, b):
    fi = hs.shape[1]
    fo = W.shape[1]
    grid = (N // _R,)
    return pl.pallas_call(
        _layer_hs2_body,
        grid=grid,
        in_specs=[
            pl.BlockSpec((_R, fi), lambda i: (i, 0)),
            pl.BlockSpec((_R, fi), lambda i: (i, 0)),
            pl.BlockSpec((_R, fi), lambda i: (i, 0)),
            pl.BlockSpec((_R, 1), lambda i: (i, 0)),
            pl.BlockSpec((fi, fo), lambda i: (0, 0)),
            pl.BlockSpec((1, fo), lambda i: (0, 0)),
        ],
        out_specs=[
            pl.BlockSpec((_R, F), lambda i: (i, 0)),
            pl.BlockSpec((_R, F), lambda i: (i, 0)),
        ],
        out_shape=[
            jax.ShapeDtypeStruct((N, F), jnp.float32),
            jax.ShapeDtypeStruct((N, F), jnp.float32),
        ],
    )(s0, s1, hs, dinv, W, b)


def _layer3_body(a0, a1, ha, b0, b1, hb, dv, wa, wb, b, h_out):
    d = dv[...]
    ma = d * (a0[...] + a1[...] + ha[...])
    mb = d * (b0[...] + b1[...] + hb[...])
    z = (
        jnp.dot(ma, wa[...], preferred_element_type=jnp.float32)
        + jnp.dot(mb, wb[...], preferred_element_type=jnp.float32)
        + b[...]
    )
    h_out[...] = jnp.maximum(z, 0.0)


def _layer3(a0, a1, ha, b0, b1, hb, dinv, Wa, Wb, b):
    fo = Wa.shape[1]
    grid = (N // _R,)
    return pl.pallas_call(
        _layer3_body,
        grid=grid,
        in_specs=[
            pl.BlockSpec((_R, F), lambda i: (i, 0)),
            pl.BlockSpec((_R, F), lambda i: (i, 0)),
            pl.BlockSpec((_R, F), lambda i: (i, 0)),
            pl.BlockSpec((_R, F), lambda i: (i, 0)),
            pl.BlockSpec((_R, F), lambda i: (i, 0)),
            pl.BlockSpec((_R, F), lambda i: (i, 0)),
            pl.BlockSpec((_R, 1), lambda i: (i, 0)),
            pl.BlockSpec((F, fo), lambda i: (0, 0)),
            pl.BlockSpec((F, fo), lambda i: (0, 0)),
            pl.BlockSpec((1, fo), lambda i: (0, 0)),
        ],
        out_specs=pl.BlockSpec((_R, fo), lambda i: (i, 0)),
        out_shape=jax.ShapeDtypeStruct((N, fo), jnp.float32),
    )(a0, a1, ha, b0, b1, hb, dinv, Wa, Wb, b)


def _pool_body(h, ids, wf1, bf1, wf2, bf2, out, pooled):
    i = pl.program_id(0)

    @pl.when(i == 0)
    def _():
        pooled[...] = jnp.full((64, 512), -jnp.inf, jnp.float32)

    hb = h[...]
    idb = ids[...]
    gmin = jnp.min(idb)
    gmax = jnp.max(idb)

    def body(g, _):
        @pl.when((g >= gmin) & (g <= gmax))
        def _():
            m = idb == g
            v = jnp.max(jnp.where(m, hb, -jnp.inf), axis=0, keepdims=True)
            pooled[pl.ds(g, 1), :] = jnp.maximum(pooled[pl.ds(g, 1), :], v)

        return 0

    lax.fori_loop(0, 64, body, 0)

    @pl.when(i == pl.num_programs(0) - 1)
    def _():
        p = pooled[...]
        g1 = jnp.dot(p, wf1[...], preferred_element_type=jnp.float32) + bf1[...]
        g1 = jnp.maximum(g1, 0.0)
        out[...] = jnp.dot(g1, wf2[...], preferred_element_type=jnp.float32) + bf2[...]


def _pool_mlp(h3, ids, Wf1, bf1, Wf2, bf2):
    grid = (N // _R,)
    return pl.pallas_call(
        _pool_body,
        grid=grid,
        in_specs=[
            pl.BlockSpec((_R, 512), lambda i: (i, 0)),
            pl.BlockSpec((_R, 1), lambda i: (i, 0)),
            pl.BlockSpec((512, 1024), lambda i: (0, 0)),
            pl.BlockSpec((1, 1024), lambda i: (0, 0)),
            pl.BlockSpec((1024, 128), lambda i: (0, 0)),
            pl.BlockSpec((1, 128), lambda i: (0, 0)),
        ],
        out_specs=pl.BlockSpec((64, 128), lambda i: (0, 0)),
        out_shape=jax.ShapeDtypeStruct((64, 128), jnp.float32),
        scratch_shapes=[pltpu.VMEM((64, 512), jnp.float32)],
    )(h3, ids, Wf1, bf1, Wf2, bf2)


def kernel(x, edge_index, batch, target, W1, b1, W2, b2, W3, b3, Wf1, bf1, Wf2, bf2):
    src = edge_index[0].astype(jnp.int32).reshape(NW, NGRP, GC, K)
    dst = edge_index[1].astype(jnp.int32).reshape(NW, NGRP, GC, K)

    dacc = _deg_call(dst)
    dinv, hs0 = _prologue(dacc, x)

    s1 = _agg_call(hs0, src, dst)
    hs1 = _layer_hs(s1[0], s1[1], hs0, dinv, W1, b1.reshape(1, -1))

    s2 = _agg_call(hs1, src, dst)
    hsa, hsb = _layer_hs2(s2[0], s2[1], hs1, dinv, W2, b2.reshape(1, -1))

    s3a = _agg_call(hsa, src, dst)
    s3b = _agg_call(hsb, src, dst)
    h3 = _layer3(
        s3a[0], s3a[1], hsa, s3b[0], s3b[1], hsb, dinv,
        W3[:F], W3[F:], b3.reshape(1, -1),
    )

    return _pool_mlp(
        h3,
        batch.astype(jnp.int32).reshape(N, 1),
        Wf1,
        bf1.reshape(1, -1),
        Wf2,
        bf2.reshape(1, -1),
    )


# speedup vs baseline: 20.8096x; 1.0023x over previous
import functools

import jax
import jax.numpy as jnp
from jax import lax
from jax.experimental import pallas as pl
from jax.experimental.pallas import tpu as pltpu
from jax.experimental.pallas import tpu_sc as plsc

N = 10000
E = 320000
NC, NS = 2, 16
NW = NC * NS
EPW = E // NW
K = 80
NCHUNK = EPW // K
NGRP, GC = 5, 25
RPT = 624
RPL = N - RPT * (NS - 1)
F = 128

_mesh = plsc.VectorSubcoreMesh(
    core_axis_name="c", subcore_axis_name="s", num_cores=NC, num_subcores=NS
)


@functools.partial(
    pl.kernel,
    out_type=jax.ShapeDtypeStruct((NC, N, 16), jnp.float32),
    mesh=_mesh,
    scratch_types=[
        pltpu.VMEM_SHARED((N, 16), jnp.float32),
        pltpu.VMEM((NGRP, GC, K), jnp.int32),
        pltpu.VMEM((K, 16), jnp.float32),
        pltpu.VMEM((RPL, 16), jnp.float32),
    ],
)
def _deg_call(dst_hbm, out_hbm, acc, dstv, onesv, zbuf):
    cid = lax.axis_index("c")
    sid = lax.axis_index("s")
    wid = cid * NS + sid

    def fill(i, _):
        zbuf[i, :] = jnp.zeros((16,), jnp.float32)
        return 0

    lax.fori_loop(0, RPL, fill, 0)

    def fill1(i, _):
        onesv[i, :] = jnp.ones((16,), jnp.float32)
        return 0

    lax.fori_loop(0, K, fill1, 0)

    base = sid * RPT
    pltpu.sync_copy(zbuf, acc.at[pl.ds(base, RPL)])
    pltpu.sync_copy(dst_hbm.at[wid], dstv)
    plsc.subcore_barrier()

    def body(g, _):
        def inner(j, _):
            pltpu.sync_copy(onesv, acc.at[dstv.at[g, j]], add=True)
            return 0

        lax.fori_loop(0, GC, inner, 0)
        return 0

    lax.fori_loop(0, NGRP, body, 0)
    plsc.subcore_barrier()

    @pl.when(sid == NS - 1)
    def _():
        pltpu.sync_copy(acc.at[pl.ds(base, RPL)], out_hbm.at[cid, pl.ds(base, RPL)])

    @pl.when(sid != NS - 1)
    def _():
        pltpu.sync_copy(acc.at[pl.ds(base, RPT)], out_hbm.at[cid, pl.ds(base, RPT)])


@functools.partial(
    pl.kernel,
    out_type=jax.ShapeDtypeStruct((NC, N, F), jnp.float32),
    mesh=_mesh,
    scratch_types=[
        pltpu.VMEM_SHARED((N, F), jnp.float32),
        pltpu.VMEM((GC, K), jnp.int32),
        pltpu.VMEM((GC, K), jnp.int32),
        pltpu.VMEM((K, F), jnp.float32),
        pltpu.VMEM((K, F), jnp.float32),
        pltpu.VMEM((K, F), jnp.float32),
        pltpu.VMEM((K, F), jnp.float32),
        pltpu.SemaphoreType.DMA,
        pltpu.SemaphoreType.DMA,
        pltpu.SemaphoreType.DMA,
        pltpu.SemaphoreType.DMA,
    ],
)
def _agg_call(
    table_hbm, src_hbm, dst_hbm, out_hbm, acc, srcv, dstv,
    rows, rows_b, rows_c, rows_d, sem, sem_b, sem_c, sem_d,
):
    cid = lax.axis_index("c")
    sid = lax.axis_index("s")
    wid = cid * NS + sid

    def fill(i, _):
        for f in range(F // 16):
            rows[i, pl.ds(f * 16, 16)] = jnp.zeros((16,), jnp.float32)
        return 0

    lax.fori_loop(0, K, fill, 0)

    base = sid * RPT
    for r in range(0, RPL, K):
        pltpu.sync_copy(rows, acc.at[pl.ds(base + r, K)])

    plsc.subcore_barrier()

    q = GC // 4

    def body(g, _):
        pltpu.sync_copy(src_hbm.at[wid, g], srcv)
        pltpu.sync_copy(dst_hbm.at[wid, g], dstv)

        def inner(j, _):
            a = pltpu.async_copy(table_hbm.at[srcv.at[j]], rows, sem)
            b = pltpu.async_copy(table_hbm.at[srcv.at[j + q]], rows_b, sem_b)
            c = pltpu.async_copy(table_hbm.at[srcv.at[j + 2 * q]], rows_c, sem_c)
            d = pltpu.async_copy(table_hbm.at[srcv.at[j + 3 * q]], rows_d, sem_d)
            a.wait()
            pltpu.sync_copy(rows, acc.at[dstv.at[j]], add=True)
            b.wait()
            pltpu.sync_copy(rows_b, acc.at[dstv.at[j + q]], add=True)
            c.wait()
            pltpu.sync_copy(rows_c, acc.at[dstv.at[j + 2 * q]], add=True)
            d.wait()
            pltpu.sync_copy(rows_d, acc.at[dstv.at[j + 3 * q]], add=True)
            return 0

        lax.fori_loop(0, q, inner, 0)
        pltpu.async_copy(table_hbm.at[srcv.at[GC - 1]], rows, sem).wait()
        pltpu.sync_copy(rows, acc.at[dstv.at[GC - 1]], add=True)
        return 0

    lax.fori_loop(0, NGRP, body, 0)
    plsc.subcore_barrier()

    @pl.when(sid == NS - 1)
    def _():
        pltpu.sync_copy(acc.at[pl.ds(base, RPL)], out_hbm.at[cid, pl.ds(base, RPL)])

    @pl.when(sid != NS - 1)
    def _():
        pltpu.sync_copy(acc.at[pl.ds(base, RPT)], out_hbm.at[cid, pl.ds(base, RPT)])


_R = 1000


def _prologue_body(d0, d1, x, dinv_out, hs_out):
    deg = d0[:, 0:1] + d1[:, 0:1] + 1.0
    dv = lax.rsqrt(deg)
    dinv_out[...] = dv
    hs_out[...] = dv * x[...]


def _prologue(dacc, x):
    grid = (N // _R,)
    return pl.pallas_call(
        _prologue_body,
        grid=grid,
        in_specs=[
            pl.BlockSpec((_R, 16), lambda i: (i, 0)),
            pl.BlockSpec((_R, 16), lambda i: (i, 0)),
            pl.BlockSpec((_R, 128), lambda i: (i, 0)),
        ],
        out_specs=[
            pl.BlockSpec((_R, 1), lambda i: (i, 0)),
            pl.BlockSpec((_R, 128), lambda i: (i, 0)),
        ],
        out_shape=[
            jax.ShapeDtypeStruct((N, 1), jnp.float32),
            jax.ShapeDtypeStruct((N, 128), jnp.float32),
        ],
    )(dacc[0], dacc[1], x)


def _layer_hs_body(s0, s1, hs, dv, w, b, hs_out):
    d = dv[...]
    m = d * (s0[...] + s1[...] + hs[...])
    z = jnp.dot(m, w[...], preferred_element_type=jnp.float32) + b[...]
    hs_out[...] = d * jnp.maximum(z, 0.0)


def _layer_hs(s0, s1, hs, dinv, W, b):
    fi = hs.shape[1]
    fo = W.shape[1]
    grid = (N // _R,)
    return pl.pallas_call(
        _layer_hs_body,
        grid=grid,
        in_specs=[
            pl.BlockSpec((_R, fi), lambda i: (i, 0)),
            pl.BlockSpec((_R, fi), lambda i: (i, 0)),
            pl.BlockSpec((_R, fi), lambda i: (i, 0)),
            pl.BlockSpec((_R, 1), lambda i: (i, 0)),
            pl.BlockSpec((fi, fo), lambda i: (0, 0)),
            pl.BlockSpec((1, fo), lambda i: (0, 0)),
        ],
        out_specs=pl.BlockSpec((_R, fo), lambda i: (i, 0)),
        out_shape=jax.ShapeDtypeStruct((N, fo), jnp.float32),
    )(s0, s1, hs, dinv, W, b)


def _layer_hs2_body(s0, s1, hs, dv, wa, wb, ba, bb, a_out, b_out):
    d = dv[...]
    m = d * (s0[...] + s1[...] + hs[...])
    za = jnp.dot(m, wa[...], preferred_element_type=jnp.float32) + ba[...]
    a_out[...] = d * jnp.maximum(za, 0.0)
    zb = jnp.dot(m, wb[...], preferred_element_type=jnp.float32) + bb[...]
    b_out[...] = d * jnp.maximum(zb, 0.0)


def _layer_hs2(s0, s1, hs, dinv, W, b):
    fi = hs.shape[1]
    grid = (N // _R,)
    return pl.pallas_call(
        _layer_hs2_body,
        grid=grid,
        in_specs=[
            pl.BlockSpec((_R, fi), lambda i: (i, 0)),
            pl.BlockSpec((_R, fi), lambda i: (i, 0)),
            pl.BlockSpec((_R, fi), lambda i: (i, 0)),
            pl.BlockSpec((_R, 1), lambda i: (i, 0)),
            pl.BlockSpec((fi, F), lambda i: (0, 0)),
            pl.BlockSpec((fi, F), lambda i: (0, 0)),
            pl.BlockSpec((1, F), lambda i: (0, 0)),
            pl.BlockSpec((1, F), lambda i: (0, 0)),
        ],
        out_specs=[
            pl.BlockSpec((_R, F), lambda i: (i, 0)),
            pl.BlockSpec((_R, F), lambda i: (i, 0)),
        ],
        out_shape=[
            jax.ShapeDtypeStruct((N, F), jnp.float32),
            jax.ShapeDtypeStruct((N, F), jnp.float32),
        ],
    )(s0, s1, hs, dinv, W[:, :F], W[:, F:], b[:, :F], b[:, F:])


def _layer3_body(a0, a1, ha, b0, b1, hb, dv, wa, wb, b, h_out):
    d = dv[...]
    ma = d * (a0[...] + a1[...] + ha[...])
    mb = d * (b0[...] + b1[...] + hb[...])
    z = (
        jnp.dot(ma, wa[...], preferred_element_type=jnp.float32)
        + jnp.dot(mb, wb[...], preferred_element_type=jnp.float32)
        + b[...]
    )
    h_out[...] = jnp.maximum(z, 0.0)


def _layer3(a0, a1, ha, b0, b1, hb, dinv, Wa, Wb, b):
    fo = Wa.shape[1]
    grid = (N // _R,)
    return pl.pallas_call(
        _layer3_body,
        grid=grid,
        in_specs=[
            pl.BlockSpec((_R, F), lambda i: (i, 0)),
            pl.BlockSpec((_R, F), lambda i: (i, 0)),
            pl.BlockSpec((_R, F), lambda i: (i, 0)),
            pl.BlockSpec((_R, F), lambda i: (i, 0)),
            pl.BlockSpec((_R, F), lambda i: (i, 0)),
            pl.BlockSpec((_R, F), lambda i: (i, 0)),
            pl.BlockSpec((_R, 1), lambda i: (i, 0)),
            pl.BlockSpec((F, fo), lambda i: (0, 0)),
            pl.BlockSpec((F, fo), lambda i: (0, 0)),
            pl.BlockSpec((1, fo), lambda i: (0, 0)),
        ],
        out_specs=pl.BlockSpec((_R, fo), lambda i: (i, 0)),
        out_shape=jax.ShapeDtypeStruct((N, fo), jnp.float32),
    )(a0, a1, ha, b0, b1, hb, dinv, Wa, Wb, b)


def _pool_body(h, ids, wf1, bf1, wf2, bf2, out, pooled):
    i = pl.program_id(0)

    @pl.when(i == 0)
    def _():
        pooled[...] = jnp.full((64, 512), -jnp.inf, jnp.float32)

    hb = h[...]
    idb = ids[...]
    gmin = jnp.min(idb)
    gmax = jnp.max(idb)

    def body(g, _):
        @pl.when((g >= gmin) & (g <= gmax))
        def _():
            m = idb == g
            v = jnp.max(jnp.where(m, hb, -jnp.inf), axis=0, keepdims=True)
            pooled[pl.ds(g, 1), :] = jnp.maximum(pooled[pl.ds(g, 1), :], v)

        return 0

    lax.fori_loop(0, 64, body, 0)

    @pl.when(i == pl.num_programs(0) - 1)
    def _():
        p = pooled[...]
        g1 = jnp.dot(p, wf1[...], preferred_element_type=jnp.float32) + bf1[...]
        g1 = jnp.maximum(g1, 0.0)
        out[...] = jnp.dot(g1, wf2[...], preferred_element_type=jnp.float32) + bf2[...]


def _pool_mlp(h3, ids, Wf1, bf1, Wf2, bf2):
    grid = (N // _R,)
    return pl.pallas_call(
        _pool_body,
        grid=grid,
        in_specs=[
            pl.BlockSpec((_R, 512), lambda i: (i, 0)),
            pl.BlockSpec((_R, 1), lambda i: (i, 0)),
            pl.BlockSpec((512, 1024), lambda i: (0, 0)),
            pl.BlockSpec((1, 1024), lambda i: (0, 0)),
            pl.BlockSpec((1024, 128), lambda i: (0, 0)),
            pl.BlockSpec((1, 128), lambda i: (0, 0)),
        ],
        out_specs=pl.BlockSpec((64, 128), lambda i: (0, 0)),
        out_shape=jax.ShapeDtypeStruct((64, 128), jnp.float32),
        scratch_shapes=[pltpu.VMEM((64, 512), jnp.float32)],
    )(h3, ids, Wf1, bf1, Wf2, bf2)


def kernel(x, edge_index, batch, target, W1, b1, W2, b2, W3, b3, Wf1, bf1, Wf2, bf2):
    src = edge_index[0].astype(jnp.int32).reshape(NW, NGRP, GC, K)
    dst = edge_index[1].astype(jnp.int32).reshape(NW, NGRP, GC, K)

    dacc = _deg_call(dst)
    dinv, hs0 = _prologue(dacc, x)

    s1 = _agg_call(hs0, src, dst)
    hs1 = _layer_hs(s1[0], s1[1], hs0, dinv, W1, b1.reshape(1, -1))

    s2 = _agg_call(hs1, src, dst)
    hsa, hsb = _layer_hs2(s2[0], s2[1], hs1, dinv, W2, b2.reshape(1, -1))

    s3a = _agg_call(hsa, src, dst)
    s3b = _agg_call(hsb, src, dst)
    h3 = _layer3(
        s3a[0], s3a[1], hsa, s3b[0], s3b[1], hsb, dinv,
        W3[:F], W3[F:], b3.reshape(1, -1),
    )

    return _pool_mlp(
        h3,
        batch.astype(jnp.int32).reshape(N, 1),
        Wf1,
        bf1.reshape(1, -1),
        Wf2,
        bf2.reshape(1, -1),
    )


# speedup vs baseline: 21.0787x; 1.0129x over previous
import functools

import jax
import jax.numpy as jnp
from jax import lax
from jax.experimental import pallas as pl
from jax.experimental.pallas import tpu as pltpu
from jax.experimental.pallas import tpu_sc as plsc

N = 10000
E = 320000
NC, NS = 2, 16
NW = NC * NS
EPW = E // NW
K = 80
NCHUNK = EPW // K
NGRP, GC = 5, 25
RPT = 624
RPL = N - RPT * (NS - 1)
F = 128

_mesh = plsc.VectorSubcoreMesh(
    core_axis_name="c", subcore_axis_name="s", num_cores=NC, num_subcores=NS
)


@functools.partial(
    pl.kernel,
    out_type=jax.ShapeDtypeStruct((NC, N, 16), jnp.float32),
    mesh=_mesh,
    scratch_types=[
        pltpu.VMEM_SHARED((N, 16), jnp.float32),
        pltpu.VMEM((NGRP, GC, K), jnp.int32),
        pltpu.VMEM((K, 16), jnp.float32),
        pltpu.VMEM((RPL, 16), jnp.float32),
    ],
)
def _deg_call(dst_hbm, out_hbm, acc, dstv, onesv, zbuf):
    cid = lax.axis_index("c")
    sid = lax.axis_index("s")
    wid = cid * NS + sid

    def fill(i, _):
        zbuf[i, :] = jnp.zeros((16,), jnp.float32)
        return 0

    lax.fori_loop(0, RPL, fill, 0)

    def fill1(i, _):
        onesv[i, :] = jnp.ones((16,), jnp.float32)
        return 0

    lax.fori_loop(0, K, fill1, 0)

    base = sid * RPT
    pltpu.sync_copy(zbuf, acc.at[pl.ds(base, RPL)])
    pltpu.sync_copy(dst_hbm.at[wid], dstv)
    plsc.subcore_barrier()

    def body(g, _):
        def inner(j, _):
            pltpu.sync_copy(onesv, acc.at[dstv.at[g, j]], add=True)
            return 0

        lax.fori_loop(0, GC, inner, 0)
        return 0

    lax.fori_loop(0, NGRP, body, 0)
    plsc.subcore_barrier()

    @pl.when(sid == NS - 1)
    def _():
        pltpu.sync_copy(acc.at[pl.ds(base, RPL)], out_hbm.at[cid, pl.ds(base, RPL)])

    @pl.when(sid != NS - 1)
    def _():
        pltpu.sync_copy(acc.at[pl.ds(base, RPT)], out_hbm.at[cid, pl.ds(base, RPT)])


@functools.partial(
    pl.kernel,
    out_type=jax.ShapeDtypeStruct((NC, N, F), jnp.float32),
    mesh=_mesh,
    scratch_types=[
        pltpu.VMEM_SHARED((N, F), jnp.float32),
        pltpu.VMEM((GC, K), jnp.int32),
        pltpu.VMEM((GC, K), jnp.int32),
        pltpu.VMEM((K, F), jnp.float32),
        pltpu.VMEM((K, F), jnp.float32),
        pltpu.VMEM((K, F), jnp.float32),
        pltpu.VMEM((K, F), jnp.float32),
        pltpu.SemaphoreType.DMA,
        pltpu.SemaphoreType.DMA,
        pltpu.SemaphoreType.DMA,
        pltpu.SemaphoreType.DMA,
    ],
)
def _agg_call(
    table_hbm, src_hbm, dst_hbm, out_hbm, acc, srcv, dstv,
    rows, rows_b, rows_c, rows_d, sem, sem_b, sem_c, sem_d,
):
    cid = lax.axis_index("c")
    sid = lax.axis_index("s")
    wid = cid * NS + sid

    def fill(i, _):
        for f in range(F // 16):
            rows[i, pl.ds(f * 16, 16)] = jnp.zeros((16,), jnp.float32)
        return 0

    lax.fori_loop(0, K, fill, 0)

    base = sid * RPT
    for r in range(0, RPL, K):
        pltpu.sync_copy(rows, acc.at[pl.ds(base + r, K)])

    plsc.subcore_barrier()

    q = GC // 4

    def body(g, _):
        pltpu.sync_copy(src_hbm.at[wid, g], srcv)
        pltpu.sync_copy(dst_hbm.at[wid, g], dstv)

        def inner(j, _):
            g0 = pltpu.async_copy(table_hbm.at[srcv.at[j]], rows, sem)
            g1 = pltpu.async_copy(table_hbm.at[srcv.at[j + q]], rows_b, sem_b)
            g2 = pltpu.async_copy(table_hbm.at[srcv.at[j + 2 * q]], rows_c, sem_c)
            g3 = pltpu.async_copy(table_hbm.at[srcv.at[j + 3 * q]], rows_d, sem_d)
            g0.wait()
            pltpu.sync_copy(rows, acc.at[dstv.at[j]], add=True)
            g1.wait()
            pltpu.sync_copy(rows_b, acc.at[dstv.at[j + q]], add=True)
            g2.wait()
            pltpu.sync_copy(rows_c, acc.at[dstv.at[j + 2 * q]], add=True)
            g3.wait()
            pltpu.sync_copy(rows_d, acc.at[dstv.at[j + 3 * q]], add=True)
            return 0

        lax.fori_loop(0, q, inner, 0)
        pltpu.async_copy(table_hbm.at[srcv.at[GC - 1]], rows, sem).wait()
        pltpu.sync_copy(rows, acc.at[dstv.at[GC - 1]], add=True)
        return 0

    lax.fori_loop(0, NGRP, body, 0)
    plsc.subcore_barrier()

    @pl.when(sid == NS - 1)
    def _():
        pltpu.sync_copy(acc.at[pl.ds(base, RPL)], out_hbm.at[cid, pl.ds(base, RPL)])

    @pl.when(sid != NS - 1)
    def _():
        pltpu.sync_copy(acc.at[pl.ds(base, RPT)], out_hbm.at[cid, pl.ds(base, RPT)])


_R = 1000


def _prologue_body(d0, d1, x, dinv_out, hs_out):
    deg = d0[:, 0:1] + d1[:, 0:1] + 1.0
    dv = lax.rsqrt(deg)
    dinv_out[...] = dv
    hs_out[...] = dv * x[...]


def _prologue(dacc, x):
    grid = (N // _R,)
    return pl.pallas_call(
        _prologue_body,
        grid=grid,
        in_specs=[
            pl.BlockSpec((_R, 16), lambda i: (i, 0)),
            pl.BlockSpec((_R, 16), lambda i: (i, 0)),
            pl.BlockSpec((_R, 128), lambda i: (i, 0)),
        ],
        out_specs=[
            pl.BlockSpec((_R, 1), lambda i: (i, 0)),
            pl.BlockSpec((_R, 128), lambda i: (i, 0)),
        ],
        out_shape=[
            jax.ShapeDtypeStruct((N, 1), jnp.float32),
            jax.ShapeDtypeStruct((N, 128), jnp.float32),
        ],
    )(dacc[0], dacc[1], x)


def _layer_hs_body(s0, s1, hs, dv, w, b, hs_out):
    d = dv[...]
    m = d * (s0[...] + s1[...] + hs[...])
    z = jnp.dot(m, w[...], preferred_element_type=jnp.float32) + b[...]
    hs_out[...] = d * jnp.maximum(z, 0.0)


def _layer_hs(s0, s1, hs, dinv, W, b):
    fi = hs.shape[1]
    fo = W.shape[1]
    grid = (N // _R,)
    return pl.pallas_call(
        _layer_hs_body,
        grid=grid,
        in_specs=[
            pl.BlockSpec((_R, fi), lambda i: (i, 0)),
            pl.BlockSpec((_R, fi), lambda i: (i, 0)),
            pl.BlockSpec((_R, fi), lambda i: (i, 0)),
            pl.BlockSpec((_R, 1), lambda i: (i, 0)),
            pl.BlockSpec((fi, fo), lambda i: (0, 0)),
            pl.BlockSpec((1, fo), lambda i: (0, 0)),
        ],
        out_specs=pl.BlockSpec((_R, fo), lambda i: (i, 0)),
        out_shape=jax.ShapeDtypeStruct((N, fo), jnp.float32),
    )(s0, s1, hs, dinv, W, b)


def _layer_hs2_body(s0, s1, hs, dv, wa, wb, ba, bb, a_out, b_out):
    d = dv[...]
    m = d * (s0[...] + s1[...] + hs[...])
    za = jnp.dot(m, wa[...], preferred_element_type=jnp.float32) + ba[...]
    a_out[...] = d * jnp.maximum(za, 0.0)
    zb = jnp.dot(m, wb[...], preferred_element_type=jnp.float32) + bb[...]
    b_out[...] = d * jnp.maximum(zb, 0.0)


def _layer_hs2(s0, s1, hs, dinv, W, b):
    fi = hs.shape[1]
    grid = (N // _R,)
    return pl.pallas_call(
        _layer_hs2_body,
        grid=grid,
        in_specs=[
            pl.BlockSpec((_R, fi), lambda i: (i, 0)),
            pl.BlockSpec((_R, fi), lambda i: (i, 0)),
            pl.BlockSpec((_R, fi), lambda i: (i, 0)),
            pl.BlockSpec((_R, 1), lambda i: (i, 0)),
            pl.BlockSpec((fi, F), lambda i: (0, 0)),
            pl.BlockSpec((fi, F), lambda i: (0, 0)),
            pl.BlockSpec((1, F), lambda i: (0, 0)),
            pl.BlockSpec((1, F), lambda i: (0, 0)),
        ],
        out_specs=[
            pl.BlockSpec((_R, F), lambda i: (i, 0)),
            pl.BlockSpec((_R, F), lambda i: (i, 0)),
        ],
        out_shape=[
            jax.ShapeDtypeStruct((N, F), jnp.float32),
            jax.ShapeDtypeStruct((N, F), jnp.float32),
        ],
    )(s0, s1, hs, dinv, W[:, :F], W[:, F:], b[:, :F], b[:, F:])


def _layer3_pool_body(
    a0, a1, ha, b0, b1, hb, dv, wa, wb, b, ids, wf1, bf1, wf2, bf2, out, pooled
):
    i = pl.program_id(0)
    d = dv[...]
    ma = d * (a0[...] + a1[...] + ha[...])
    mb = d * (b0[...] + b1[...] + hb[...])
    z = (
        jnp.dot(ma, wa[...], preferred_element_type=jnp.float32)
        + jnp.dot(mb, wb[...], preferred_element_type=jnp.float32)
        + b[...]
    )
    z = jnp.maximum(z, 0.0)

    @pl.when(i == 0)
    def _():
        pooled[...] = jnp.full((64, 512), -jnp.inf, jnp.float32)

    idb = ids[...]
    gmin = jnp.min(idb)
    gmax = jnp.max(idb)

    def body(g, _):
        @pl.when((g >= gmin) & (g <= gmax))
        def _():
            m = idb == g
            v = jnp.max(jnp.where(m, z, -jnp.inf), axis=0, keepdims=True)
            pooled[pl.ds(g, 1), :] = jnp.maximum(pooled[pl.ds(g, 1), :], v)

        return 0

    lax.fori_loop(0, 64, body, 0)

    @pl.when(i == pl.num_programs(0) - 1)
    def _():
        p = pooled[...]
        g1 = jnp.dot(p, wf1[...], preferred_element_type=jnp.float32) + bf1[...]
        g1 = jnp.maximum(g1, 0.0)
        out[...] = jnp.dot(g1, wf2[...], preferred_element_type=jnp.float32) + bf2[...]


def _layer3_pool(a0, a1, ha, b0, b1, hb, dinv, Wa, Wb, b, ids, Wf1, bf1, Wf2, bf2):
    fo = Wa.shape[1]
    grid = (N // _R,)
    return pl.pallas_call(
        _layer3_pool_body,
        grid=grid,
        in_specs=[
            pl.BlockSpec((_R, F), lambda i: (i, 0)),
            pl.BlockSpec((_R, F), lambda i: (i, 0)),
            pl.BlockSpec((_R, F), lambda i: (i, 0)),
            pl.BlockSpec((_R, F), lambda i: (i, 0)),
            pl.BlockSpec((_R, F), lambda i: (i, 0)),
            pl.BlockSpec((_R, F), lambda i: (i, 0)),
            pl.BlockSpec((_R, 1), lambda i: (i, 0)),
            pl.BlockSpec((F, fo), lambda i: (0, 0)),
            pl.BlockSpec((F, fo), lambda i: (0, 0)),
            pl.BlockSpec((1, fo), lambda i: (0, 0)),
            pl.BlockSpec((_R, 1), lambda i: (i, 0)),
            pl.BlockSpec((512, 1024), lambda i: (0, 0)),
            pl.BlockSpec((1, 1024), lambda i: (0, 0)),
            pl.BlockSpec((1024, 128), lambda i: (0, 0)),
            pl.BlockSpec((1, 128), lambda i: (0, 0)),
        ],
        out_specs=pl.BlockSpec((64, 128), lambda i: (0, 0)),
        out_shape=jax.ShapeDtypeStruct((64, 128), jnp.float32),
        scratch_shapes=[pltpu.VMEM((64, 512), jnp.float32)],
    )(a0, a1, ha, b0, b1, hb, dinv, Wa, Wb, b, ids, Wf1, bf1, Wf2, bf2)


def kernel(x, edge_index, batch, target, W1, b1, W2, b2, W3, b3, Wf1, bf1, Wf2, bf2):
    src = edge_index[0].astype(jnp.int32).reshape(NW, NGRP, GC, K)
    dst = edge_index[1].astype(jnp.int32).reshape(NW, NGRP, GC, K)

    dacc = _deg_call(dst)
    dinv, hs0 = _prologue(dacc, x)

    s1 = _agg_call(hs0, src, dst)
    hs1 = _layer_hs(s1[0], s1[1], hs0, dinv, W1, b1.reshape(1, -1))

    s2 = _agg_call(hs1, src, dst)
    hsa, hsb = _layer_hs2(s2[0], s2[1], hs1, dinv, W2, b2.reshape(1, -1))

    s3a = _agg_call(hsa, src, dst)
    s3b = _agg_call(hsb, src, dst)
    return _layer3_pool(
        s3a[0], s3a[1], hsa, s3b[0], s3b[1], hsb, dinv,
        W3[:F], W3[F:], b3.reshape(1, -1),
        batch.astype(jnp.int32).reshape(N, 1),
        Wf1, bf1.reshape(1, -1), Wf2, bf2.reshape(1, -1),
    )


# speedup vs baseline: 21.3487x; 1.0128x over previous
import functools

import jax
import jax.numpy as jnp
from jax import lax
from jax.experimental import pallas as pl
from jax.experimental.pallas import tpu as pltpu
from jax.experimental.pallas import tpu_sc as plsc

N = 10000
E = 320000
NC, NS = 2, 16
NW = NC * NS
EPW = E // NW
K = 80
NCHUNK = EPW // K
NGRP, GC = 5, 25
RPT = 624
RPL = N - RPT * (NS - 1)
F = 128

_mesh = plsc.VectorSubcoreMesh(
    core_axis_name="c", subcore_axis_name="s", num_cores=NC, num_subcores=NS
)


@functools.partial(
    pl.kernel,
    out_type=jax.ShapeDtypeStruct((NC, N, 16), jnp.float32),
    mesh=_mesh,
    scratch_types=[
        pltpu.VMEM_SHARED((N, 16), jnp.float32),
        pltpu.VMEM((NGRP, GC, K), jnp.int32),
        pltpu.VMEM((K, 16), jnp.float32),
        pltpu.VMEM((RPL, 16), jnp.float32),
    ],
)
def _deg_call(dst_hbm, out_hbm, acc, dstv, onesv, zbuf):
    cid = lax.axis_index("c")
    sid = lax.axis_index("s")
    wid = cid * NS + sid

    def fill(i, _):
        zbuf[i, :] = jnp.zeros((16,), jnp.float32)
        return 0

    lax.fori_loop(0, RPL, fill, 0)

    def fill1(i, _):
        onesv[i, :] = jnp.ones((16,), jnp.float32)
        return 0

    lax.fori_loop(0, K, fill1, 0)

    base = sid * RPT
    pltpu.sync_copy(zbuf, acc.at[pl.ds(base, RPL)])
    pltpu.sync_copy(dst_hbm.at[wid], dstv)
    plsc.subcore_barrier()

    def body(g, _):
        def inner(j, _):
            pltpu.sync_copy(onesv, acc.at[dstv.at[g, j]], add=True)
            return 0

        lax.fori_loop(0, GC, inner, 0)
        return 0

    lax.fori_loop(0, NGRP, body, 0)
    plsc.subcore_barrier()

    @pl.when(sid == NS - 1)
    def _():
        pltpu.sync_copy(acc.at[pl.ds(base, RPL)], out_hbm.at[cid, pl.ds(base, RPL)])

    @pl.when(sid != NS - 1)
    def _():
        pltpu.sync_copy(acc.at[pl.ds(base, RPT)], out_hbm.at[cid, pl.ds(base, RPT)])


@functools.partial(
    pl.kernel,
    out_type=jax.ShapeDtypeStruct((NC, N, F), jnp.float32),
    mesh=_mesh,
    scratch_types=[
        pltpu.VMEM_SHARED((N, F), jnp.float32),
        pltpu.VMEM((2, GC, K), jnp.int32),
        pltpu.VMEM((K, F), jnp.float32),
        pltpu.VMEM((K, F), jnp.float32),
        pltpu.VMEM((K, F), jnp.float32),
        pltpu.VMEM((K, F), jnp.float32),
        pltpu.SemaphoreType.DMA,
        pltpu.SemaphoreType.DMA,
        pltpu.SemaphoreType.DMA,
        pltpu.SemaphoreType.DMA,
    ],
)
def _agg_call(
    table_hbm, sd_hbm, out_hbm, acc, sdv,
    rows, rows_b, rows_c, rows_d, sem, sem_b, sem_c, sem_d,
):
    cid = lax.axis_index("c")
    sid = lax.axis_index("s")
    wid = cid * NS + sid

    def fill(i, _):
        for f in range(F // 16):
            rows[i, pl.ds(f * 16, 16)] = jnp.zeros((16,), jnp.float32)
        return 0

    lax.fori_loop(0, K, fill, 0)

    base = sid * RPT
    for r in range(0, RPL, K):
        pltpu.sync_copy(rows, acc.at[pl.ds(base + r, K)])

    plsc.subcore_barrier()

    q = GC // 4

    def body(g, _):
        pltpu.sync_copy(sd_hbm.at[wid, g], sdv)

        def inner(j, _):
            g0 = pltpu.async_copy(table_hbm.at[sdv.at[0, j]], rows, sem)
            g1 = pltpu.async_copy(table_hbm.at[sdv.at[0, j + q]], rows_b, sem_b)
            g2 = pltpu.async_copy(table_hbm.at[sdv.at[0, j + 2 * q]], rows_c, sem_c)
            g3 = pltpu.async_copy(table_hbm.at[sdv.at[0, j + 3 * q]], rows_d, sem_d)
            g0.wait()
            pltpu.sync_copy(rows, acc.at[sdv.at[1, j]], add=True)
            g1.wait()
            pltpu.sync_copy(rows_b, acc.at[sdv.at[1, j + q]], add=True)
            g2.wait()
            pltpu.sync_copy(rows_c, acc.at[sdv.at[1, j + 2 * q]], add=True)
            g3.wait()
            pltpu.sync_copy(rows_d, acc.at[sdv.at[1, j + 3 * q]], add=True)
            return 0

        lax.fori_loop(0, q, inner, 0)
        pltpu.async_copy(table_hbm.at[sdv.at[0, GC - 1]], rows, sem).wait()
        pltpu.sync_copy(rows, acc.at[sdv.at[1, GC - 1]], add=True)
        return 0

    lax.fori_loop(0, NGRP, body, 0)
    plsc.subcore_barrier()

    @pl.when(sid == NS - 1)
    def _():
        pltpu.sync_copy(acc.at[pl.ds(base, RPL)], out_hbm.at[cid, pl.ds(base, RPL)])

    @pl.when(sid != NS - 1)
    def _():
        pltpu.sync_copy(acc.at[pl.ds(base, RPT)], out_hbm.at[cid, pl.ds(base, RPT)])


_R = 1000


def _prologue_body(d0, d1, x, dinv_out, hs_out):
    deg = d0[:, 0:1] + d1[:, 0:1] + 1.0
    dv = lax.rsqrt(deg)
    dinv_out[...] = dv
    hs_out[...] = dv * x[...]


def _prologue(dacc, x):
    grid = (N // _R,)
    return pl.pallas_call(
        _prologue_body,
        grid=grid,
        in_specs=[
            pl.BlockSpec((_R, 16), lambda i: (i, 0)),
            pl.BlockSpec((_R, 16), lambda i: (i, 0)),
            pl.BlockSpec((_R, 128), lambda i: (i, 0)),
        ],
        out_specs=[
            pl.BlockSpec((_R, 1), lambda i: (i, 0)),
            pl.BlockSpec((_R, 128), lambda i: (i, 0)),
        ],
        out_shape=[
            jax.ShapeDtypeStruct((N, 1), jnp.float32),
            jax.ShapeDtypeStruct((N, 128), jnp.float32),
        ],
    )(dacc[0], dacc[1], x)


def _layer_hs_body(s0, s1, hs, dv, w, b, hs_out):
    d = dv[...]
    m = d * (s0[...] + s1[...] + hs[...])
    z = jnp.dot(m, w[...], preferred_element_type=jnp.float32) + b[...]
    hs_out[...] = d * jnp.maximum(z, 0.0)


def _layer_hs(s0, s1, hs, dinv, W, b):
    fi = hs.shape[1]
    fo = W.shape[1]
    grid = (N // _R,)
    return pl.pallas_call(
        _layer_hs_body,
        grid=grid,
        in_specs=[
            pl.BlockSpec((_R, fi), lambda i: (i, 0)),
            pl.BlockSpec((_R, fi), lambda i: (i, 0)),
            pl.BlockSpec((_R, fi), lambda i: (i, 0)),
            pl.BlockSpec((_R, 1), lambda i: (i, 0)),
            pl.BlockSpec((fi, fo), lambda i: (0, 0)),
            pl.BlockSpec((1, fo), lambda i: (0, 0)),
        ],
        out_specs=pl.BlockSpec((_R, fo), lambda i: (i, 0)),
        out_shape=jax.ShapeDtypeStruct((N, fo), jnp.float32),
    )(s0, s1, hs, dinv, W, b)


def _layer_hs2_body(s0, s1, hs, dv, wa, wb, ba, bb, a_out, b_out):
    d = dv[...]
    m = d * (s0[...] + s1[...] + hs[...])
    za = jnp.dot(m, wa[...], preferred_element_type=jnp.float32) + ba[...]
    a_out[...] = d * jnp.maximum(za, 0.0)
    zb = jnp.dot(m, wb[...], preferred_element_type=jnp.float32) + bb[...]
    b_out[...] = d * jnp.maximum(zb, 0.0)


def _layer_hs2(s0, s1, hs, dinv, W, b):
    fi = hs.shape[1]
    grid = (N // _R,)
    return pl.pallas_call(
        _layer_hs2_body,
        grid=grid,
        in_specs=[
            pl.BlockSpec((_R, fi), lambda i: (i, 0)),
            pl.BlockSpec((_R, fi), lambda i: (i, 0)),
            pl.BlockSpec((_R, fi), lambda i: (i, 0)),
            pl.BlockSpec((_R, 1), lambda i: (i, 0)),
            pl.BlockSpec((fi, F), lambda i: (0, 0)),
            pl.BlockSpec((fi, F), lambda i: (0, 0)),
            pl.BlockSpec((1, F), lambda i: (0, 0)),
            pl.BlockSpec((1, F), lambda i: (0, 0)),
        ],
        out_specs=[
            pl.BlockSpec((_R, F), lambda i: (i, 0)),
            pl.BlockSpec((_R, F), lambda i: (i, 0)),
        ],
        out_shape=[
            jax.ShapeDtypeStruct((N, F), jnp.float32),
            jax.ShapeDtypeStruct((N, F), jnp.float32),
        ],
    )(s0, s1, hs, dinv, W[:, :F], W[:, F:], b[:, :F], b[:, F:])


def _layer3_pool_body(
    a0, a1, ha, b0, b1, hb, dv, wa, wb, b, ids, wf1, bf1, wf2, bf2, out, pooled
):
    i = pl.program_id(0)
    d = dv[...]
    ma = d * (a0[...] + a1[...] + ha[...])
    mb = d * (b0[...] + b1[...] + hb[...])
    z = (
        jnp.dot(ma, wa[...], preferred_element_type=jnp.float32)
        + jnp.dot(mb, wb[...], preferred_element_type=jnp.float32)
        + b[...]
    )
    z = jnp.maximum(z, 0.0)

    @pl.when(i == 0)
    def _():
        pooled[...] = jnp.full((64, 512), -jnp.inf, jnp.float32)

    idb = ids[...]
    gmin = jnp.min(idb)
    gmax = jnp.max(idb)

    def body(g, _):
        @pl.when((g >= gmin) & (g <= gmax))
        def _():
            m = idb == g
            v = jnp.max(jnp.where(m, z, -jnp.inf), axis=0, keepdims=True)
            pooled[pl.ds(g, 1), :] = jnp.maximum(pooled[pl.ds(g, 1), :], v)

        return 0

    lax.fori_loop(0, 64, body, 0)

    @pl.when(i == pl.num_programs(0) - 1)
    def _():
        p = pooled[...]
        g1 = jnp.dot(p, wf1[...], preferred_element_type=jnp.float32) + bf1[...]
        g1 = jnp.maximum(g1, 0.0)
        out[...] = jnp.dot(g1, wf2[...], preferred_element_type=jnp.float32) + bf2[...]


def _layer3_pool(a0, a1, ha, b0, b1, hb, dinv, Wa, Wb, b, ids, Wf1, bf1, Wf2, bf2):
    fo = Wa.shape[1]
    grid = (N // _R,)
    return pl.pallas_call(
        _layer3_pool_body,
        grid=grid,
        in_specs=[
            pl.BlockSpec((_R, F), lambda i: (i, 0)),
            pl.BlockSpec((_R, F), lambda i: (i, 0)),
            pl.BlockSpec((_R, F), lambda i: (i, 0)),
            pl.BlockSpec((_R, F), lambda i: (i, 0)),
            pl.BlockSpec((_R, F), lambda i: (i, 0)),
            pl.BlockSpec((_R, F), lambda i: (i, 0)),
            pl.BlockSpec((_R, 1), lambda i: (i, 0)),
            pl.BlockSpec((F, fo), lambda i: (0, 0)),
            pl.BlockSpec((F, fo), lambda i: (0, 0)),
            pl.BlockSpec((1, fo), lambda i: (0, 0)),
            pl.BlockSpec((_R, 1), lambda i: (i, 0)),
            pl.BlockSpec((512, 1024), lambda i: (0, 0)),
            pl.BlockSpec((1, 1024), lambda i: (0, 0)),
            pl.BlockSpec((1024, 128), lambda i: (0, 0)),
            pl.BlockSpec((1, 128), lambda i: (0, 0)),
        ],
        out_specs=pl.BlockSpec((64, 128), lambda i: (0, 0)),
        out_shape=jax.ShapeDtypeStruct((64, 128), jnp.float32),
        scratch_shapes=[pltpu.VMEM((64, 512), jnp.float32)],
    )(a0, a1, ha, b0, b1, hb, dinv, Wa, Wb, b, ids, Wf1, bf1, Wf2, bf2)


def kernel(x, edge_index, batch, target, W1, b1, W2, b2, W3, b3, Wf1, bf1, Wf2, bf2):
    src = edge_index[0].astype(jnp.int32).reshape(NW, NGRP, GC, K)
    dst = edge_index[1].astype(jnp.int32).reshape(NW, NGRP, GC, K)
    sd = jnp.stack([src, dst], axis=2)

    dacc = _deg_call(dst)
    dinv, hs0 = _prologue(dacc, x)

    s1 = _agg_call(hs0, sd)
    hs1 = _layer_hs(s1[0], s1[1], hs0, dinv, W1, b1.reshape(1, -1))

    s2 = _agg_call(hs1, sd)
    hsa, hsb = _layer_hs2(s2[0], s2[1], hs1, dinv, W2, b2.reshape(1, -1))

    s3a = _agg_call(hsa, sd)
    s3b = _agg_call(hsb, sd)
    return _layer3_pool(
        s3a[0], s3a[1], hsa, s3b[0], s3b[1], hsb, dinv,
        W3[:F], W3[F:], b3.reshape(1, -1),
        batch.astype(jnp.int32).reshape(N, 1),
        Wf1, bf1.reshape(1, -1), Wf2, bf2.reshape(1, -1),
    )
